# baseline (speedup 1.0000x reference)
import functools

import jax
import jax.numpy as jnp
from jax import lax
from jax.experimental import pallas as pl
from jax.experimental.pallas import tpu as pltpu
from jax.experimental.pallas import tpu_sc as plsc

N_NODES = 10000
N_EDGES = 320000
OUT_CH = 128

NC = 2
NS = 16
L = 16
NW = NC * NS

E_PER_TILE = N_EDGES // NW
CH = 64
NCHUNK = 160
SLOTS = NCHUNK * CH
D = 5


def _prep_body(x_ref, w_ref, asrc_ref, adst_ref, h_ref, as_ref, ad_ref,
               gm_ref):
    h = jnp.dot(x_ref[...], w_ref[...], preferred_element_type=jnp.float32)
    h_ref[...] = h
    a_s = jnp.sum(h * asrc_ref[...], axis=-1, keepdims=True)
    a_d = jnp.sum(h * adst_ref[...], axis=-1, keepdims=True)
    as_ref[...] = a_s
    ad_ref[...] = a_d
    g = jnp.max(a_s) + jnp.max(a_d)
    gm_ref[0, 0] = jnp.where(g >= 0.0, g, 0.2 * g)


def _prep_call(x, W, att_src, att_dst):
    return pl.pallas_call(
        _prep_body,
        out_shape=[
            jax.ShapeDtypeStruct((N_NODES, OUT_CH), jnp.float32),
            jax.ShapeDtypeStruct((N_NODES, 1), jnp.float32),
            jax.ShapeDtypeStruct((N_NODES, 1), jnp.float32),
            jax.ShapeDtypeStruct((1, 1), jnp.float32),
        ],
        out_specs=[
            pl.BlockSpec(memory_space=pltpu.VMEM),
            pl.BlockSpec(memory_space=pltpu.VMEM),
            pl.BlockSpec(memory_space=pltpu.VMEM),
            pl.BlockSpec(memory_space=pltpu.SMEM),
        ],
    )(x, W, att_src, att_dst)


_mesh = plsc.VectorSubcoreMesh(core_axis_name="c", subcore_axis_name="s",
                               num_cores=NC, num_subcores=NS)


@functools.partial(
    pl.kernel,
    out_type=[
        jax.ShapeDtypeStruct((NC, N_NODES, OUT_CH), jnp.float32),
        jax.ShapeDtypeStruct((NC * N_NODES,), jnp.float32),
    ],
    mesh=_mesh,
    compiler_params=pltpu.CompilerParams(needs_layout_passes=False),
    scratch_types=[
        pltpu.VMEM((L,), jnp.float32),
        pltpu.VMEM((D, CH), jnp.int32),
        pltpu.VMEM((D, CH), jnp.int32),
        pltpu.VMEM((D, CH), jnp.float32),
        pltpu.VMEM((D, CH), jnp.float32),
        pltpu.VMEM((D, CH), jnp.float32),
        pltpu.VMEM((D, CH, OUT_CH), jnp.float32),
        pltpu.VMEM((1024,), jnp.float32),
        pltpu.VMEM_SHARED((N_NODES, OUT_CH), jnp.float32),
        pltpu.VMEM_SHARED((N_NODES,), jnp.float32),
        pltpu.SemaphoreType.DMA((D,)),
        pltpu.SemaphoreType.DMA((D,)),
        pltpu.SemaphoreType.DMA((D,)),
        pltpu.SemaphoreType.DMA((D,)),
    ],
)
def _sc_edges(src_hbm, dst_hbm, as_hbm, ad_hbm, gm_hbm, h_hbm,
              num_hbm, den_hbm,
              gm_l, srcb, dstb, asb, adb, wb, rowsb, zb1,
              num_sh, den_sh, isem, gsem, ssem, dsem):
    cid = lax.axis_index("c")
    sid = lax.axis_index("s")
    wid = cid * NS + sid
    ebase = wid * SLOTS

    pltpu.sync_copy(gm_hbm, gm_l)
    gm_vec = gm_l[...]

    zero16 = jnp.zeros((L,), jnp.float32)

    @pl.loop(0, 1024 // L)
    def _z1(i):
        zb1[pl.ds(i * L, L)] = zero16

    @pl.loop(0, CH)
    def _zrow(i):
        for j in range(OUT_CH // L):
            rowsb[0, i, pl.ds(j * L, L)] = zero16

    zbase = sid * 624
    for off in range(0, 640, CH):
        pltpu.sync_copy(rowsb.at[0], num_sh.at[pl.ds(zbase + off, CH)])

    @pl.when(sid < 10)
    def _zden():
        pltpu.sync_copy(zb1.at[pl.ds(0, 1000)],
                        den_sh.at[pl.ds(sid * 1000, 1000)])

    def load_idx(m, p):
        pltpu.async_copy(src_hbm.at[pl.ds(ebase + m * CH, CH)],
                         srcb.at[p], isem.at[p])
        pltpu.async_copy(dst_hbm.at[pl.ds(ebase + m * CH, CH)],
                         dstb.at[p], isem.at[p])

    def wait_idx(p):
        pltpu.make_async_copy(src_hbm.at[pl.ds(0, CH)], srcb.at[p],
                              isem.at[p]).wait()
        pltpu.make_async_copy(dst_hbm.at[pl.ds(0, CH)], dstb.at[p],
                              isem.at[p]).wait()

    def issue_gathers(p):
        pltpu.async_copy(as_hbm.at[srcb.at[p]], asb.at[p], gsem.at[p])
        pltpu.async_copy(ad_hbm.at[dstb.at[p]], adb.at[p], gsem.at[p])
        pltpu.async_copy(h_hbm.at[srcb.at[p]], rowsb.at[p], gsem.at[p])

    def wait_gathers(p):
        pltpu.make_async_copy(as_hbm.at[srcb.at[p]], asb.at[p],
                              gsem.at[p]).wait()
        pltpu.make_async_copy(ad_hbm.at[dstb.at[p]], adb.at[p],
                              gsem.at[p]).wait()
        pltpu.make_async_copy(h_hbm.at[srcb.at[p]], rowsb.at[p],
                              gsem.at[p]).wait()

    def drain_scatters(m, p):
        pltpu.make_async_copy(rowsb.at[p], num_sh.at[dstb.at[p]],
                              ssem.at[p]).wait()
        pltpu.make_async_copy(wb.at[p], den_sh.at[dstb.at[p]],
                              dsem.at[p]).wait()

    load_idx(0, 0)
    load_idx(1, 1)
    wait_idx(0)
    wait_idx(1)
    issue_gathers(0)
    issue_gathers(1)
    load_idx(2, 2)

    plsc.subcore_barrier()

    @pl.loop(0, NCHUNK, step=D)
    def _main(c0):
        for b in range(D):
            m = c0 + b
            p_drain = (b + 3) % D
            p_idx = (b + 3) % D
            p_gath = (b + 2) % D

            @pl.when(m >= 2)
            def _drain():
                drain_scatters(m - 2, p_drain)

            @pl.when(m + 3 < NCHUNK)
            def _idx():
                load_idx(m + 3, p_idx)

            @pl.when(m + 2 < NCHUNK)
            def _gath():
                wait_idx(p_gath)
                issue_gathers(p_gath)

            wait_gathers(b)

            for v in range(CH // L):
                a_s = asb[b, pl.ds(v * L, L)]
                a_d = adb[b, pl.ds(v * L, L)]
                e = a_s + a_d
                e = jnp.where(e >= 0.0, e, 0.2 * e)
                w = jnp.exp(e - gm_vec)
                ids = m * CH + v * L + lax.iota(jnp.int32, L)
                wb[b, pl.ds(v * L, L)] = jnp.where(ids < E_PER_TILE, w, 0.0)

            @pl.loop(0, CH // L)
            def _scale(v):
                wv = wb[b, pl.ds(v * L, L)]
                for jj in range(L):
                    ws = wv[jj]
                    row = v * L + jj
                    for j in range(OUT_CH // L):
                        rowsb[b, row, pl.ds(j * L, L)] = (
                            rowsb[b, row, pl.ds(j * L, L)] * ws)

            pltpu.async_copy(rowsb.at[b], num_sh.at[dstb.at[b]],
                             ssem.at[b], add=True)
            pltpu.async_copy(wb.at[b], den_sh.at[dstb.at[b]],
                             dsem.at[b], add=True)

    for m in (NCHUNK - 2, NCHUNK - 1):
        drain_scatters(m, m % D)

    plsc.subcore_barrier()

    @pl.when(sid < 15)
    def _dnum():
        pltpu.sync_copy(num_sh.at[pl.ds(sid * 632, 632)],
                        num_hbm.at[cid, pl.ds(sid * 632, 632)])

    @pl.when(sid == 15)
    def _dnum_last():
        pltpu.sync_copy(num_sh.at[pl.ds(9480, 520)],
                        num_hbm.at[cid, pl.ds(9480, 520)])

    @pl.when(sid < 10)
    def _dden():
        pltpu.sync_copy(den_sh.at[pl.ds(sid * 1000, 1000)],
                        zb1.at[pl.ds(0, 1000)])
        pltpu.sync_copy(zb1.at[pl.ds(0, 1000)],
                        den_hbm.at[pl.ds(cid * N_NODES + sid * 1000, 1000)])


def _finish_body(num_ref, den_ref, b_ref, out_ref):
    s = num_ref[0] + num_ref[1]
    d = den_ref[0] + den_ref[1] + 1e-16
    out_ref[...] = jnp.maximum(s / d + b_ref[...], 0.0)


def _finish_call(num, den, b):
    return pl.pallas_call(
        _finish_body,
        out_shape=jax.ShapeDtypeStruct((N_NODES, OUT_CH), jnp.float32),
    )(num, den, b)


@jax.jit
def kernel(x, edge_index, W, att_src, att_dst, b):
    src = edge_index[0].astype(jnp.int32)
    dst = edge_index[1].astype(jnp.int32)
    pad = SLOTS - E_PER_TILE
    srcf = jnp.pad(src.reshape(NW, E_PER_TILE),
                   ((0, 0), (0, pad))).reshape(NW * SLOTS)
    dstf = jnp.pad(dst.reshape(NW, E_PER_TILE),
                   ((0, 0), (0, pad))).reshape(NW * SLOTS)
    h, a_s, a_d, gm = _prep_call(x, W, att_src.reshape(1, OUT_CH),
                                 att_dst.reshape(1, OUT_CH))
    gm16 = jnp.broadcast_to(gm.reshape(1), (L,))
    num, den = _sc_edges(srcf, dstf, a_s.reshape(N_NODES),
                         a_d.reshape(N_NODES), gm16, h)
    out = _finish_call(num, den.reshape(NC, N_NODES, 1), b.reshape(1, OUT_CH))
    return out


# speedup vs baseline: 1.6212x; 1.6212x over previous
import functools

import jax
import jax.numpy as jnp
from jax import lax
from jax.experimental import pallas as pl
from jax.experimental.pallas import tpu as pltpu
from jax.experimental.pallas import tpu_sc as plsc

N_NODES = 10000
N_EDGES = 320000
OUT_CH = 128

NC = 2
NS = 16
L = 16
NW = NC * NS

E_PER_TILE = N_EDGES // NW
CH = 112
NCHUNK = 90
SLOTS = NCHUNK * CH
D = 3
DI = 6


def _prep_body(x_ref, w_ref, asrc_ref, adst_ref, h_ref, as_ref, ad_ref,
               gm_ref):
    h = jnp.dot(x_ref[...], w_ref[...], preferred_element_type=jnp.float32)
    h_ref[...] = h
    a_s = jnp.sum(h * asrc_ref[...], axis=-1, keepdims=True)
    a_d = jnp.sum(h * adst_ref[...], axis=-1, keepdims=True)
    as_ref[...] = a_s
    ad_ref[...] = a_d
    g = jnp.max(a_s) + jnp.max(a_d)
    gm_ref[0, 0] = jnp.where(g >= 0.0, g, 0.2 * g)


def _prep_call(x, W, att_src, att_dst):
    return pl.pallas_call(
        _prep_body,
        out_shape=[
            jax.ShapeDtypeStruct((N_NODES, OUT_CH), jnp.float32),
            jax.ShapeDtypeStruct((N_NODES, 1), jnp.float32),
            jax.ShapeDtypeStruct((N_NODES, 1), jnp.float32),
            jax.ShapeDtypeStruct((1, 1), jnp.float32),
        ],
        out_specs=[
            pl.BlockSpec(memory_space=pltpu.VMEM),
            pl.BlockSpec(memory_space=pltpu.VMEM),
            pl.BlockSpec(memory_space=pltpu.VMEM),
            pl.BlockSpec(memory_space=pltpu.SMEM),
        ],
    )(x, W, att_src, att_dst)


_mesh = plsc.VectorSubcoreMesh(core_axis_name="c", subcore_axis_name="s",
                               num_cores=NC, num_subcores=NS)


@functools.partial(
    pl.kernel,
    out_type=[
        jax.ShapeDtypeStruct((NC, N_NODES, OUT_CH), jnp.float32),
        jax.ShapeDtypeStruct((NC * N_NODES,), jnp.float32),
    ],
    mesh=_mesh,
    compiler_params=pltpu.CompilerParams(needs_layout_passes=False),
    scratch_types=[
        pltpu.VMEM((L,), jnp.float32),
        pltpu.VMEM((DI, CH), jnp.int32),
        pltpu.VMEM((DI, CH), jnp.int32),
        pltpu.VMEM((D, CH), jnp.float32),
        pltpu.VMEM((D, CH), jnp.float32),
        pltpu.VMEM((D, CH), jnp.float32),
        pltpu.VMEM((D, CH, OUT_CH), jnp.float32),
        pltpu.VMEM((640,), jnp.float32),
        pltpu.VMEM((512,), jnp.float32),
        pltpu.VMEM_SHARED((N_NODES, OUT_CH), jnp.float32),
        pltpu.VMEM_SHARED((N_NODES,), jnp.float32),
        pltpu.SemaphoreType.DMA((DI,)),
        pltpu.SemaphoreType.DMA((D,)),
        pltpu.SemaphoreType.DMA((D,)),
        pltpu.SemaphoreType.DMA((D,)),
    ],
)
def _sc_edges(src_hbm, dst_hbm, as_hbm, ad_hbm, gm_hbm, h_hbm,
              num_hbm, den_hbm,
              gm_l, srcb, dstb, asb, adb, wb, rowsb, stage, zb1,
              num_sh, den_sh, isem, gsem, ssem, dsem):
    cid = lax.axis_index("c")
    sid = lax.axis_index("s")
    wid = cid * NS + sid
    ebase = wid * SLOTS

    pltpu.sync_copy(gm_hbm, gm_l)
    gm_vec = gm_l[...]

    tbase = sid * 624

    zero16 = jnp.zeros((L,), jnp.float32)

    @pl.loop(0, 512 // L)
    def _z1(i):
        zb1[pl.ds(i * L, L)] = zero16

    @pl.loop(0, CH)
    def _zrow(i):
        for j in range(OUT_CH // L):
            rowsb[0, i, pl.ds(j * L, L)] = zero16

    for off in (0, 112, 224, 336, 448, 528):
        pltpu.sync_copy(rowsb.at[0], num_sh.at[pl.ds(tbase + off, CH)])

    @pl.when(sid < 10)
    def _zden():
        pltpu.sync_copy(zb1.at[pl.ds(0, 512)],
                        den_sh.at[pl.ds(sid * 1000, 512)])
        pltpu.sync_copy(zb1.at[pl.ds(0, 488)],
                        den_sh.at[pl.ds(sid * 1000 + 512, 488)])

    def load_idx(m, p):
        pltpu.async_copy(src_hbm.at[pl.ds(ebase + m * CH, CH)],
                         srcb.at[p], isem.at[p])
        pltpu.async_copy(dst_hbm.at[pl.ds(ebase + m * CH, CH)],
                         dstb.at[p], isem.at[p])

    def wait_idx(p):
        pltpu.make_async_copy(src_hbm.at[pl.ds(0, CH)], srcb.at[p],
                              isem.at[p]).wait()
        pltpu.make_async_copy(dst_hbm.at[pl.ds(0, CH)], dstb.at[p],
                              isem.at[p]).wait()

    def issue_gathers(p6, p3):
        pltpu.async_copy(as_hbm.at[srcb.at[p6]], asb.at[p3], gsem.at[p3])
        pltpu.async_copy(ad_hbm.at[dstb.at[p6]], adb.at[p3], gsem.at[p3])
        pltpu.async_copy(h_hbm.at[srcb.at[p6]], rowsb.at[p3], gsem.at[p3])

    def wait_gathers(p6, p3):
        pltpu.make_async_copy(as_hbm.at[srcb.at[p6]], asb.at[p3],
                              gsem.at[p3]).wait()
        pltpu.make_async_copy(ad_hbm.at[dstb.at[p6]], adb.at[p3],
                              gsem.at[p3]).wait()
        pltpu.make_async_copy(h_hbm.at[srcb.at[p6]], rowsb.at[p3],
                              gsem.at[p3]).wait()

    def drain_scatters(p6, p3):
        pltpu.make_async_copy(rowsb.at[p3], num_sh.at[dstb.at[p6]],
                              ssem.at[p3]).wait()
        pltpu.make_async_copy(wb.at[p3], den_sh.at[dstb.at[p6]],
                              dsem.at[p3]).wait()

    load_idx(0, 0)
    load_idx(1, 1)
    load_idx(2, 2)
    plsc.subcore_barrier()
    wait_idx(0)
    issue_gathers(0, 0)

    @pl.loop(0, NCHUNK, step=DI)
    def _main(c0):
        for b in range(DI):
            m = c0 + b
            b3 = b % D

            @pl.when(m >= 2)
            def _drain():
                drain_scatters((b + 4) % DI, (b + 1) % D)

            @pl.when(m + 3 < NCHUNK)
            def _idx():
                load_idx(m + 3, (b + 3) % DI)

            @pl.when(m + 1 < NCHUNK)
            def _gath():
                wait_idx((b + 1) % DI)
                issue_gathers((b + 1) % DI, (b + 1) % D)

            wait_gathers(b, b3)

            for v in range(CH // L):
                a_s = asb[b3, pl.ds(v * L, L)]
                a_d = adb[b3, pl.ds(v * L, L)]
                e = a_s + a_d
                e = jnp.where(e >= 0.0, e, 0.2 * e)
                w = jnp.exp(e - gm_vec)
                ids = m * CH + v * L + lax.iota(jnp.int32, L)
                wb[b3, pl.ds(v * L, L)] = jnp.where(ids < E_PER_TILE, w, 0.0)

            @pl.loop(0, CH // L)
            def _scale(v):
                wv = wb[b3, pl.ds(v * L, L)]
                for jj in range(L):
                    ws = wv[jj]
                    row = v * L + jj
                    for j in range(OUT_CH // L):
                        rowsb[b3, row, pl.ds(j * L, L)] = (
                            rowsb[b3, row, pl.ds(j * L, L)] * ws)

            pltpu.async_copy(rowsb.at[b3], num_sh.at[dstb.at[b]],
                             ssem.at[b3], add=True)
            pltpu.async_copy(wb.at[b3], den_sh.at[dstb.at[b]],
                             dsem.at[b3], add=True)

    for m in (NCHUNK - 2, NCHUNK - 1):
        drain_scatters(m % DI, m % D)

    plsc.subcore_barrier()

    @pl.when(sid < 15)
    def _dnum():
        pltpu.sync_copy(num_sh.at[pl.ds(sid * 632, 632)],
                        num_hbm.at[cid, pl.ds(sid * 632, 632)])

    @pl.when(sid == 15)
    def _dnum_last():
        pltpu.sync_copy(num_sh.at[pl.ds(9480, 520)],
                        num_hbm.at[cid, pl.ds(9480, 520)])

    @pl.when(sid < 10)
    def _dden():
        pltpu.sync_copy(den_sh.at[pl.ds(sid * 1000, 640)],
                        stage.at[pl.ds(0, 640)])
        pltpu.sync_copy(den_sh.at[pl.ds(sid * 1000 + 640, 360)],
                        zb1.at[pl.ds(0, 360)])
        pltpu.sync_copy(stage.at[pl.ds(0, 640)],
                        den_hbm.at[pl.ds(cid * N_NODES + sid * 1000, 640)])
        pltpu.sync_copy(zb1.at[pl.ds(0, 360)],
                        den_hbm.at[pl.ds(cid * N_NODES + sid * 1000 + 640,
                                         360)])


def _finish_body(num_ref, den_ref, b_ref, out_ref):
    s = num_ref[0] + num_ref[1]
    d = den_ref[0] + den_ref[1] + 1e-16
    out_ref[...] = jnp.maximum(s / d + b_ref[...], 0.0)


def _finish_call(num, den, b):
    return pl.pallas_call(
        _finish_body,
        out_shape=jax.ShapeDtypeStruct((N_NODES, OUT_CH), jnp.float32),
    )(num, den, b)


@jax.jit
def kernel(x, edge_index, W, att_src, att_dst, b):
    src = edge_index[0].astype(jnp.int32)
    dst = edge_index[1].astype(jnp.int32)
    pad = SLOTS - E_PER_TILE
    srcf = jnp.pad(src.reshape(NW, E_PER_TILE),
                   ((0, 0), (0, pad))).reshape(NW * SLOTS)
    dstf = jnp.pad(dst.reshape(NW, E_PER_TILE),
                   ((0, 0), (0, pad))).reshape(NW * SLOTS)
    h, a_s, a_d, gm = _prep_call(x, W, att_src.reshape(1, OUT_CH),
                                 att_dst.reshape(1, OUT_CH))
    gm16 = jnp.broadcast_to(gm.reshape(1), (L,))
    num, den = _sc_edges(srcf, dstf, a_s.reshape(N_NODES),
                         a_d.reshape(N_NODES), gm16, h)
    out = _finish_call(num, den.reshape(NC, N_NODES, 1), b.reshape(1, OUT_CH))
    return out


# speedup vs baseline: 2.4370x; 1.5032x over previous
import functools

import jax
import jax.numpy as jnp
from jax import lax
from jax.experimental import pallas as pl
from jax.experimental.pallas import tpu as pltpu
from jax.experimental.pallas import tpu_sc as plsc

N_NODES = 10000
N_EDGES = 320000
OUT_CH = 128

NC = 2
NS = 16
L = 16
NW = NC * NS

E_PER_TILE = N_EDGES // NW
CH = 112
NCHUNK = 90
CLAMP = E_PER_TILE - CH
D = 3
DI = 6


def _prep_body(x_ref, w_ref, asrc_ref, adst_ref, h_ref, as_ref, ad_ref,
               gm_ref):
    h = jnp.dot(x_ref[...], w_ref[...], preferred_element_type=jnp.float32)
    h_ref[...] = h
    a_s = jnp.sum(h * asrc_ref[...], axis=-1, keepdims=True)
    a_d = jnp.sum(h * adst_ref[...], axis=-1, keepdims=True)
    as_ref[...] = a_s
    ad_ref[...] = a_d
    g = jnp.max(a_s) + jnp.max(a_d)
    gm_ref[0, 0] = jnp.where(g >= 0.0, g, 0.2 * g)


def _prep_call(x, W, att_src, att_dst):
    return pl.pallas_call(
        _prep_body,
        out_shape=[
            jax.ShapeDtypeStruct((N_NODES, OUT_CH), jnp.float32),
            jax.ShapeDtypeStruct((N_NODES, 1), jnp.float32),
            jax.ShapeDtypeStruct((N_NODES, 1), jnp.float32),
            jax.ShapeDtypeStruct((1, 1), jnp.float32),
        ],
        out_specs=[
            pl.BlockSpec(memory_space=pltpu.VMEM),
            pl.BlockSpec(memory_space=pltpu.VMEM),
            pl.BlockSpec(memory_space=pltpu.VMEM),
            pl.BlockSpec(memory_space=pltpu.SMEM),
        ],
    )(x, W, att_src, att_dst)


_mesh = plsc.VectorSubcoreMesh(core_axis_name="c", subcore_axis_name="s",
                               num_cores=NC, num_subcores=NS)


@functools.partial(
    pl.kernel,
    out_type=[
        jax.ShapeDtypeStruct((NC, N_NODES, OUT_CH), jnp.float32),
        jax.ShapeDtypeStruct((NC * N_NODES,), jnp.float32),
    ],
    mesh=_mesh,
    compiler_params=pltpu.CompilerParams(needs_layout_passes=False),
    scratch_types=[
        pltpu.VMEM((L,), jnp.float32),
        pltpu.VMEM((DI, CH), jnp.int32),
        pltpu.VMEM((DI, CH), jnp.int32),
        pltpu.VMEM((D, CH), jnp.float32),
        pltpu.VMEM((D, CH), jnp.float32),
        pltpu.VMEM((D, CH), jnp.float32),
        pltpu.VMEM((D, CH, OUT_CH), jnp.float32),
        pltpu.VMEM((640,), jnp.float32),
        pltpu.VMEM((512,), jnp.float32),
        pltpu.VMEM_SHARED((N_NODES, OUT_CH), jnp.float32),
        pltpu.VMEM_SHARED((N_NODES,), jnp.float32),
        pltpu.SemaphoreType.DMA((DI,)),
        pltpu.SemaphoreType.DMA((D,)),
        pltpu.SemaphoreType.DMA((D,)),
        pltpu.SemaphoreType.DMA((D,)),
        pltpu.SemaphoreType.DMA((D,)),
    ],
)
def _sc_edges(src_hbm, dst_hbm, as_hbm, ad_hbm, gm_hbm, h_hbm,
              num_hbm, den_hbm,
              gm_l, srcb, dstb, asb, adb, wb, rowsb, stage, zb1,
              num_sh, den_sh, isem, gsem, hsem, ssem, dsem):
    cid = lax.axis_index("c")
    sid = lax.axis_index("s")
    wid = cid * NS + sid
    ebase = wid * E_PER_TILE

    pltpu.sync_copy(gm_hbm, gm_l)
    gm_vec = gm_l[...]

    tbase = sid * 624

    zero16 = jnp.zeros((L,), jnp.float32)

    @pl.loop(0, 512 // L)
    def _z1(i):
        zb1[pl.ds(i * L, L)] = zero16

    @pl.loop(0, CH)
    def _zrow(i):
        for j in range(OUT_CH // L):
            rowsb[0, i, pl.ds(j * L, L)] = zero16

    for off in (0, 112, 224, 336, 448, 528):
        pltpu.sync_copy(rowsb.at[0], num_sh.at[pl.ds(tbase + off, CH)])

    @pl.when(sid < 10)
    def _zden():
        pltpu.sync_copy(zb1.at[pl.ds(0, 512)],
                        den_sh.at[pl.ds(sid * 1000, 512)])
        pltpu.sync_copy(zb1.at[pl.ds(0, 488)],
                        den_sh.at[pl.ds(sid * 1000 + 512, 488)])

    def chunk_base(m):
        return jnp.minimum(m * CH, CLAMP)

    def load_idx(m, p):
        base = ebase + chunk_base(m)
        pltpu.async_copy(src_hbm.at[pl.ds(base, CH)], srcb.at[p],
                         isem.at[p])
        pltpu.async_copy(dst_hbm.at[pl.ds(base, CH)], dstb.at[p],
                         isem.at[p])

    def wait_idx(p):
        pltpu.make_async_copy(src_hbm.at[pl.ds(0, CH)], srcb.at[p],
                              isem.at[p]).wait()
        pltpu.make_async_copy(dst_hbm.at[pl.ds(0, CH)], dstb.at[p],
                              isem.at[p]).wait()

    def issue_gathers(p6, p3):
        pltpu.async_copy(h_hbm.at[srcb.at[p6]], rowsb.at[p3], hsem.at[p3])
        pltpu.async_copy(as_hbm.at[srcb.at[p6]], asb.at[p3], gsem.at[p3])
        pltpu.async_copy(ad_hbm.at[dstb.at[p6]], adb.at[p3], gsem.at[p3])

    def wait_small_gathers(p6, p3):
        pltpu.make_async_copy(as_hbm.at[srcb.at[p6]], asb.at[p3],
                              gsem.at[p3]).wait()
        pltpu.make_async_copy(ad_hbm.at[dstb.at[p6]], adb.at[p3],
                              gsem.at[p3]).wait()

    def wait_h_gather(p6, p3):
        pltpu.make_async_copy(h_hbm.at[srcb.at[p6]], rowsb.at[p3],
                              hsem.at[p3]).wait()

    def drain_scatters(p6, p3):
        pltpu.make_async_copy(rowsb.at[p3], num_sh.at[dstb.at[p6]],
                              ssem.at[p3]).wait()
        pltpu.make_async_copy(wb.at[p3], den_sh.at[dstb.at[p6]],
                              dsem.at[p3]).wait()

    load_idx(0, 0)
    load_idx(1, 1)
    load_idx(2, 2)
    plsc.subcore_barrier()
    wait_idx(0)
    issue_gathers(0, 0)

    @pl.loop(0, NCHUNK, step=DI)
    def _main(c0):
        for b in range(DI):
            m = c0 + b
            b3 = b % D

            @pl.when(m >= 2)
            def _drain():
                drain_scatters((b + 4) % DI, (b + 1) % D)

            @pl.when(m + 3 < NCHUNK)
            def _idx():
                load_idx(m + 3, (b + 3) % DI)

            @pl.when(m + 1 < NCHUNK)
            def _gath():
                wait_idx((b + 1) % DI)
                issue_gathers((b + 1) % DI, (b + 1) % D)

            wait_small_gathers(b, b3)

            cb = chunk_base(m)
            for v in range(CH // L):
                a_s = asb[b3, pl.ds(v * L, L)]
                a_d = adb[b3, pl.ds(v * L, L)]
                e = a_s + a_d
                e = jnp.where(e >= 0.0, e, 0.2 * e)
                w = jnp.exp(e - gm_vec)
                pos = cb + v * L + lax.iota(jnp.int32, L)
                wb[b3, pl.ds(v * L, L)] = jnp.where(pos >= m * CH, w, 0.0)

            wait_h_gather(b, b3)

            @pl.loop(0, CH // L)
            def _scale(v):
                wv = wb[b3, pl.ds(v * L, L)]
                for jj in range(L):
                    ws = wv[jj]
                    row = v * L + jj
                    for j in range(OUT_CH // L):
                        rowsb[b3, row, pl.ds(j * L, L)] = (
                            rowsb[b3, row, pl.ds(j * L, L)] * ws)

            pltpu.async_copy(rowsb.at[b3], num_sh.at[dstb.at[b]],
                             ssem.at[b3], add=True)
            pltpu.async_copy(wb.at[b3], den_sh.at[dstb.at[b]],
                             dsem.at[b3], add=True)

    for m in (NCHUNK - 2, NCHUNK - 1):
        drain_scatters(m % DI, m % D)

    plsc.subcore_barrier()

    @pl.when(sid < 15)
    def _dnum():
        pltpu.sync_copy(num_sh.at[pl.ds(sid * 632, 632)],
                        num_hbm.at[cid, pl.ds(sid * 632, 632)])

    @pl.when(sid == 15)
    def _dnum_last():
        pltpu.sync_copy(num_sh.at[pl.ds(9480, 520)],
                        num_hbm.at[cid, pl.ds(9480, 520)])

    @pl.when(sid < 10)
    def _dden():
        pltpu.sync_copy(den_sh.at[pl.ds(sid * 1000, 640)],
                        stage.at[pl.ds(0, 640)])
        pltpu.sync_copy(den_sh.at[pl.ds(sid * 1000 + 640, 360)],
                        zb1.at[pl.ds(0, 360)])
        pltpu.sync_copy(stage.at[pl.ds(0, 640)],
                        den_hbm.at[pl.ds(cid * N_NODES + sid * 1000, 640)])
        pltpu.sync_copy(zb1.at[pl.ds(0, 360)],
                        den_hbm.at[pl.ds(cid * N_NODES + sid * 1000 + 640,
                                         360)])


def _finish_body(num_ref, den_ref, b_ref, out_ref):
    s = num_ref[0] + num_ref[1]
    d = den_ref[0] + den_ref[1] + 1e-16
    out_ref[...] = jnp.maximum(s / d + b_ref[...], 0.0)


def _finish_call(num, den, b):
    return pl.pallas_call(
        _finish_body,
        out_shape=jax.ShapeDtypeStruct((N_NODES, OUT_CH), jnp.float32),
    )(num, den, b)


@jax.jit
def kernel(x, edge_index, W, att_src, att_dst, b):
    src = edge_index[0].astype(jnp.int32)
    dst = edge_index[1].astype(jnp.int32)
    h, a_s, a_d, gm = _prep_call(x, W, att_src.reshape(1, OUT_CH),
                                 att_dst.reshape(1, OUT_CH))
    gm16 = jnp.broadcast_to(gm.reshape(1), (L,))
    num, den = _sc_edges(src, dst, a_s.reshape(N_NODES),
                         a_d.reshape(N_NODES), gm16, h)
    out = _finish_call(num, den.reshape(NC, N_NODES, 1), b.reshape(1, OUT_CH))
    return out


# speedup vs baseline: 2.4899x; 1.0217x over previous
import functools

import jax
import jax.numpy as jnp
from jax import lax
from jax.experimental import pallas as pl
from jax.experimental.pallas import tpu as pltpu
from jax.experimental.pallas import tpu_sc as plsc

N_NODES = 10000
N_EDGES = 320000
OUT_CH = 128

NC = 2
NS = 16
L = 16
NW = NC * NS

E_PER_TILE = N_EDGES // NW
CH = 112
NCHUNK = 90
CLAMP = E_PER_TILE - CH
D = 3
DI = 6


def _prep_body(x_ref, w_ref, asrc_ref, adst_ref, h_ref, as_ref, ad_ref,
               gm_ref):
    h = jnp.dot(x_ref[...], w_ref[...], preferred_element_type=jnp.float32)
    h_ref[...] = h
    a_s = jnp.sum(h * asrc_ref[...], axis=-1)
    a_d = jnp.sum(h * adst_ref[...], axis=-1)
    as_ref[...] = a_s
    ad_ref[...] = a_d
    g = jnp.max(a_s) + jnp.max(a_d)
    g = jnp.where(g >= 0.0, g, 0.2 * g)
    gm_ref[...] = jnp.broadcast_to(g, (L,))


def _prep_call(x, W, att_src, att_dst):
    return pl.pallas_call(
        _prep_body,
        out_shape=[
            jax.ShapeDtypeStruct((N_NODES, OUT_CH), jnp.float32),
            jax.ShapeDtypeStruct((N_NODES,), jnp.float32),
            jax.ShapeDtypeStruct((N_NODES,), jnp.float32),
            jax.ShapeDtypeStruct((L,), jnp.float32),
        ],
    )(x, W, att_src, att_dst)


_mesh = plsc.VectorSubcoreMesh(core_axis_name="c", subcore_axis_name="s",
                               num_cores=NC, num_subcores=NS)


@functools.partial(
    pl.kernel,
    out_type=[
        jax.ShapeDtypeStruct((NC, N_NODES, OUT_CH), jnp.float32),
        jax.ShapeDtypeStruct((NC * N_NODES,), jnp.float32),
    ],
    mesh=_mesh,
    compiler_params=pltpu.CompilerParams(needs_layout_passes=False),
    scratch_types=[
        pltpu.VMEM((L,), jnp.float32),
        pltpu.VMEM((DI, CH), jnp.int32),
        pltpu.VMEM((DI, CH), jnp.int32),
        pltpu.VMEM((D, CH), jnp.float32),
        pltpu.VMEM((D, CH), jnp.float32),
        pltpu.VMEM((D, CH), jnp.float32),
        pltpu.VMEM((D, CH, OUT_CH), jnp.float32),
        pltpu.VMEM((640,), jnp.float32),
        pltpu.VMEM((512,), jnp.float32),
        pltpu.VMEM_SHARED((N_NODES, OUT_CH), jnp.float32),
        pltpu.VMEM_SHARED((N_NODES,), jnp.float32),
        pltpu.SemaphoreType.DMA((DI,)),
        pltpu.SemaphoreType.DMA((D,)),
        pltpu.SemaphoreType.DMA((D,)),
        pltpu.SemaphoreType.DMA((D,)),
        pltpu.SemaphoreType.DMA((D,)),
    ],
)
def _sc_edges(src_hbm, dst_hbm, as_hbm, ad_hbm, gm_hbm, h_hbm,
              num_hbm, den_hbm,
              gm_l, srcb, dstb, asb, adb, wb, rowsb, stage, zb1,
              num_sh, den_sh, isem, gsem, hsem, ssem, dsem):
    cid = lax.axis_index("c")
    sid = lax.axis_index("s")
    wid = cid * NS + sid
    ebase = wid * E_PER_TILE

    pltpu.sync_copy(gm_hbm, gm_l)
    gm_vec = gm_l[...]

    tbase = sid * 624

    zero16 = jnp.zeros((L,), jnp.float32)

    @pl.loop(0, 512 // L)
    def _z1(i):
        zb1[pl.ds(i * L, L)] = zero16

    @pl.loop(0, CH)
    def _zrow(i):
        for j in range(OUT_CH // L):
            rowsb[0, i, pl.ds(j * L, L)] = zero16

    for off in (0, 112, 224, 336, 448, 528):
        pltpu.sync_copy(rowsb.at[0], num_sh.at[pl.ds(tbase + off, CH)])

    @pl.when(sid < 10)
    def _zden():
        pltpu.sync_copy(zb1.at[pl.ds(0, 512)],
                        den_sh.at[pl.ds(sid * 1000, 512)])
        pltpu.sync_copy(zb1.at[pl.ds(0, 488)],
                        den_sh.at[pl.ds(sid * 1000 + 512, 488)])

    def chunk_base(m):
        return jnp.minimum(m * CH, CLAMP)

    def load_idx(m, p):
        base = ebase + chunk_base(m)
        pltpu.async_copy(src_hbm.at[pl.ds(base, CH)], srcb.at[p],
                         isem.at[p])
        pltpu.async_copy(dst_hbm.at[pl.ds(base, CH)], dstb.at[p],
                         isem.at[p])

    def wait_idx(p):
        pltpu.make_async_copy(src_hbm.at[pl.ds(0, CH)], srcb.at[p],
                              isem.at[p]).wait()
        pltpu.make_async_copy(dst_hbm.at[pl.ds(0, CH)], dstb.at[p],
                              isem.at[p]).wait()

    def issue_gathers(p6, p3):
        pltpu.async_copy(h_hbm.at[srcb.at[p6]], rowsb.at[p3], hsem.at[p3])
        pltpu.async_copy(as_hbm.at[srcb.at[p6]], asb.at[p3], gsem.at[p3])
        pltpu.async_copy(ad_hbm.at[dstb.at[p6]], adb.at[p3], gsem.at[p3])

    def wait_small_gathers(p6, p3):
        pltpu.make_async_copy(as_hbm.at[srcb.at[p6]], asb.at[p3],
                              gsem.at[p3]).wait()
        pltpu.make_async_copy(ad_hbm.at[dstb.at[p6]], adb.at[p3],
                              gsem.at[p3]).wait()

    def wait_h_gather(p6, p3):
        pltpu.make_async_copy(h_hbm.at[srcb.at[p6]], rowsb.at[p3],
                              hsem.at[p3]).wait()

    def drain_scatters(p6, p3):
        pltpu.make_async_copy(rowsb.at[p3], num_sh.at[dstb.at[p6]],
                              ssem.at[p3]).wait()
        pltpu.make_async_copy(wb.at[p3], den_sh.at[dstb.at[p6]],
                              dsem.at[p3]).wait()

    load_idx(0, 0)
    load_idx(1, 1)
    load_idx(2, 2)
    plsc.subcore_barrier()
    wait_idx(0)
    issue_gathers(0, 0)

    @pl.loop(0, NCHUNK, step=DI)
    def _main(c0):
        for b in range(DI):
            m = c0 + b
            b3 = b % D

            @pl.when(m >= 2)
            def _drain():
                drain_scatters((b + 4) % DI, (b + 1) % D)

            @pl.when(m + 3 < NCHUNK)
            def _idx():
                load_idx(m + 3, (b + 3) % DI)

            @pl.when(m + 1 < NCHUNK)
            def _gath():
                wait_idx((b + 1) % DI)
                issue_gathers((b + 1) % DI, (b + 1) % D)

            wait_small_gathers(b, b3)

            cb = chunk_base(m)
            for v in range(CH // L):
                a_s = asb[b3, pl.ds(v * L, L)]
                a_d = adb[b3, pl.ds(v * L, L)]
                e = a_s + a_d
                e = jnp.where(e >= 0.0, e, 0.2 * e)
                w = jnp.exp(e - gm_vec)
                pos = cb + v * L + lax.iota(jnp.int32, L)
                wb[b3, pl.ds(v * L, L)] = jnp.where(pos >= m * CH, w, 0.0)

            wait_h_gather(b, b3)

            @pl.loop(0, CH // L)
            def _scale(v):
                wv = wb[b3, pl.ds(v * L, L)]
                for jj in range(L):
                    ws = wv[jj]
                    row = v * L + jj
                    for j in range(OUT_CH // L):
                        rowsb[b3, row, pl.ds(j * L, L)] = (
                            rowsb[b3, row, pl.ds(j * L, L)] * ws)

            pltpu.async_copy(rowsb.at[b3], num_sh.at[dstb.at[b]],
                             ssem.at[b3], add=True)
            pltpu.async_copy(wb.at[b3], den_sh.at[dstb.at[b]],
                             dsem.at[b3], add=True)

    for m in (NCHUNK - 2, NCHUNK - 1):
        drain_scatters(m % DI, m % D)

    plsc.subcore_barrier()

    @pl.when(sid < 15)
    def _dnum():
        pltpu.sync_copy(num_sh.at[pl.ds(sid * 632, 632)],
                        num_hbm.at[cid, pl.ds(sid * 632, 632)])

    @pl.when(sid == 15)
    def _dnum_last():
        pltpu.sync_copy(num_sh.at[pl.ds(9480, 520)],
                        num_hbm.at[cid, pl.ds(9480, 520)])

    @pl.when(sid < 10)
    def _dden():
        pltpu.sync_copy(den_sh.at[pl.ds(sid * 1000, 640)],
                        stage.at[pl.ds(0, 640)])
        pltpu.sync_copy(den_sh.at[pl.ds(sid * 1000 + 640, 360)],
                        zb1.at[pl.ds(0, 360)])
        pltpu.sync_copy(stage.at[pl.ds(0, 640)],
                        den_hbm.at[pl.ds(cid * N_NODES + sid * 1000, 640)])
        pltpu.sync_copy(zb1.at[pl.ds(0, 360)],
                        den_hbm.at[pl.ds(cid * N_NODES + sid * 1000 + 640,
                                         360)])


def _finish_body(num_ref, den_ref, b_ref, out_ref):
    s = num_ref[0] + num_ref[1]
    d = den_ref[0] + den_ref[1] + 1e-16
    out_ref[...] = jnp.maximum(s / d + b_ref[...], 0.0)


def _finish_call(num, den, b):
    return pl.pallas_call(
        _finish_body,
        out_shape=jax.ShapeDtypeStruct((N_NODES, OUT_CH), jnp.float32),
    )(num, den, b)


@jax.jit
def kernel(x, edge_index, W, att_src, att_dst, b):
    src = edge_index[0].astype(jnp.int32)
    dst = edge_index[1].astype(jnp.int32)
    h, a_s, a_d, gm16 = _prep_call(x, W, att_src.reshape(1, OUT_CH),
                                   att_dst.reshape(1, OUT_CH))
    num, den = _sc_edges(src, dst, a_s, a_d, gm16, h)
    out = _finish_call(num, den.reshape(NC, N_NODES, 1), b.reshape(1, OUT_CH))
    return out


# speedup vs baseline: 2.4926x; 1.0011x over previous
import functools

import jax
import jax.numpy as jnp
from jax import lax
from jax.experimental import pallas as pl
from jax.experimental.pallas import tpu as pltpu
from jax.experimental.pallas import tpu_sc as plsc

N_NODES = 10000
N_EDGES = 320000
OUT_CH = 128

NC = 2
NS = 16
L = 16
NW = NC * NS

E_PER_TILE = N_EDGES // NW
CH = 112
NCHUNK = 90
CLAMP = E_PER_TILE - CH
D = 3
DI = 6


def _prep_body(x_ref, w_ref, asrc_ref, adst_ref, h_ref, as_ref, ad_ref,
               gm_ref):
    h = jnp.dot(x_ref[...], w_ref[...], preferred_element_type=jnp.float32)
    h_ref[...] = h
    a_s = jnp.sum(h * asrc_ref[...], axis=-1)
    a_d = jnp.sum(h * adst_ref[...], axis=-1)
    as_ref[...] = a_s
    ad_ref[...] = a_d
    g = jnp.max(a_s) + jnp.max(a_d)
    g = jnp.where(g >= 0.0, g, 0.2 * g)
    gm_ref[...] = jnp.broadcast_to(g, (L,))


def _prep_call(x, W, att_src, att_dst):
    return pl.pallas_call(
        _prep_body,
        out_shape=[
            jax.ShapeDtypeStruct((N_NODES, OUT_CH), jnp.float32),
            jax.ShapeDtypeStruct((N_NODES,), jnp.float32),
            jax.ShapeDtypeStruct((N_NODES,), jnp.float32),
            jax.ShapeDtypeStruct((L,), jnp.float32),
        ],
    )(x, W, att_src, att_dst)


_mesh = plsc.VectorSubcoreMesh(core_axis_name="c", subcore_axis_name="s",
                               num_cores=NC, num_subcores=NS)


@functools.partial(
    pl.kernel,
    out_type=[
        jax.ShapeDtypeStruct((NC, N_NODES, OUT_CH), jnp.float32),
        jax.ShapeDtypeStruct((NC * N_NODES,), jnp.float32),
    ],
    mesh=_mesh,
    compiler_params=pltpu.CompilerParams(needs_layout_passes=False),
    scratch_types=[
        pltpu.VMEM((L,), jnp.float32),
        pltpu.VMEM((DI, CH), jnp.int32),
        pltpu.VMEM((DI, CH), jnp.int32),
        pltpu.VMEM((D, CH), jnp.float32),
        pltpu.VMEM((D, CH), jnp.float32),
        pltpu.VMEM((D, CH), jnp.float32),
        pltpu.VMEM((D, CH, OUT_CH), jnp.float32),
        pltpu.VMEM((640,), jnp.float32),
        pltpu.VMEM((512,), jnp.float32),
        pltpu.VMEM_SHARED((N_NODES, OUT_CH), jnp.float32),
        pltpu.VMEM_SHARED((N_NODES,), jnp.float32),
        pltpu.SemaphoreType.DMA((DI,)),
        pltpu.SemaphoreType.DMA((D,)),
        pltpu.SemaphoreType.DMA((D,)),
        pltpu.SemaphoreType.DMA((D,)),
        pltpu.SemaphoreType.DMA((D,)),
    ],
)
def _sc_edges(src_hbm, dst_hbm, as_hbm, ad_hbm, gm_hbm, h_hbm,
              num_hbm, den_hbm,
              gm_l, srcb, dstb, asb, adb, wb, rowsb, stage, zb1,
              num_sh, den_sh, isem, gsem, hsem, ssem, dsem):
    cid = lax.axis_index("c")
    sid = lax.axis_index("s")
    wid = cid * NS + sid
    ebase = wid * E_PER_TILE

    pltpu.sync_copy(gm_hbm, gm_l)
    gm_vec = gm_l[...]

    tbase = sid * 624

    zero16 = jnp.zeros((L,), jnp.float32)

    @pl.loop(0, 512 // L)
    def _z1(i):
        zb1[pl.ds(i * L, L)] = zero16

    @pl.loop(0, CH)
    def _zrow(i):
        for j in range(OUT_CH // L):
            rowsb[0, i, pl.ds(j * L, L)] = zero16

    for off in (0, 112, 224, 336, 448, 528):
        pltpu.sync_copy(rowsb.at[0], num_sh.at[pl.ds(tbase + off, CH)])

    @pl.when(sid < 10)
    def _zden():
        pltpu.sync_copy(zb1.at[pl.ds(0, 512)],
                        den_sh.at[pl.ds(sid * 1000, 512)])
        pltpu.sync_copy(zb1.at[pl.ds(0, 488)],
                        den_sh.at[pl.ds(sid * 1000 + 512, 488)])

    def chunk_base(m):
        return jnp.minimum(m * CH, CLAMP)

    def load_idx(m, p):
        base = ebase + chunk_base(m)
        pltpu.async_copy(src_hbm.at[pl.ds(base, CH)], srcb.at[p],
                         isem.at[p])
        pltpu.async_copy(dst_hbm.at[pl.ds(base, CH)], dstb.at[p],
                         isem.at[p])

    def wait_idx(p):
        pltpu.make_async_copy(src_hbm.at[pl.ds(0, CH)], srcb.at[p],
                              isem.at[p]).wait()
        pltpu.make_async_copy(dst_hbm.at[pl.ds(0, CH)], dstb.at[p],
                              isem.at[p]).wait()

    def issue_gathers(p6, p3):
        pltpu.async_copy(h_hbm.at[srcb.at[p6, pl.ds(0, CH // 2)]],
                         rowsb.at[p3, pl.ds(0, CH // 2)], hsem.at[p3])
        pltpu.async_copy(h_hbm.at[srcb.at[p6, pl.ds(CH // 2, CH // 2)]],
                         rowsb.at[p3, pl.ds(CH // 2, CH // 2)], hsem.at[p3])
        pltpu.async_copy(as_hbm.at[srcb.at[p6]], asb.at[p3], gsem.at[p3])
        pltpu.async_copy(ad_hbm.at[dstb.at[p6]], adb.at[p3], gsem.at[p3])

    def wait_small_gathers(p6, p3):
        pltpu.make_async_copy(as_hbm.at[srcb.at[p6]], asb.at[p3],
                              gsem.at[p3]).wait()
        pltpu.make_async_copy(ad_hbm.at[dstb.at[p6]], adb.at[p3],
                              gsem.at[p3]).wait()

    def wait_h_gather(p6, p3):
        pltpu.make_async_copy(h_hbm.at[srcb.at[p6, pl.ds(0, CH // 2)]],
                              rowsb.at[p3, pl.ds(0, CH // 2)],
                              hsem.at[p3]).wait()
        pltpu.make_async_copy(h_hbm.at[srcb.at[p6, pl.ds(CH // 2, CH // 2)]],
                              rowsb.at[p3, pl.ds(CH // 2, CH // 2)],
                              hsem.at[p3]).wait()

    def drain_scatters(p6, p3):
        pltpu.make_async_copy(rowsb.at[p3], num_sh.at[dstb.at[p6]],
                              ssem.at[p3]).wait()
        pltpu.make_async_copy(wb.at[p3], den_sh.at[dstb.at[p6]],
                              dsem.at[p3]).wait()

    load_idx(0, 0)
    load_idx(1, 1)
    load_idx(2, 2)
    plsc.subcore_barrier()
    wait_idx(0)
    issue_gathers(0, 0)

    @pl.loop(0, NCHUNK, step=DI)
    def _main(c0):
        for b in range(DI):
            m = c0 + b
            b3 = b % D

            @pl.when(m >= 2)
            def _drain():
                drain_scatters((b + 4) % DI, (b + 1) % D)

            @pl.when(m + 3 < NCHUNK)
            def _idx():
                load_idx(m + 3, (b + 3) % DI)

            @pl.when(m + 1 < NCHUNK)
            def _gath():
                wait_idx((b + 1) % DI)
                issue_gathers((b + 1) % DI, (b + 1) % D)

            wait_small_gathers(b, b3)

            cb = chunk_base(m)
            for v in range(CH // L):
                a_s = asb[b3, pl.ds(v * L, L)]
                a_d = adb[b3, pl.ds(v * L, L)]
                e = a_s + a_d
                e = jnp.where(e >= 0.0, e, 0.2 * e)
                w = jnp.exp(e - gm_vec)
                pos = cb + v * L + lax.iota(jnp.int32, L)
                wb[b3, pl.ds(v * L, L)] = jnp.where(pos >= m * CH, w, 0.0)

            wait_h_gather(b, b3)

            @pl.loop(0, CH // L)
            def _scale(v):
                wv = wb[b3, pl.ds(v * L, L)]
                for jj in range(L):
                    ws = wv[jj]
                    row = v * L + jj
                    for j in range(OUT_CH // L):
                        rowsb[b3, row, pl.ds(j * L, L)] = (
                            rowsb[b3, row, pl.ds(j * L, L)] * ws)

            pltpu.async_copy(rowsb.at[b3], num_sh.at[dstb.at[b]],
                             ssem.at[b3], add=True)
            pltpu.async_copy(wb.at[b3], den_sh.at[dstb.at[b]],
                             dsem.at[b3], add=True)

    for m in (NCHUNK - 2, NCHUNK - 1):
        drain_scatters(m % DI, m % D)

    plsc.subcore_barrier()

    @pl.when(sid < 15)
    def _dnum():
        pltpu.sync_copy(num_sh.at[pl.ds(sid * 632, 632)],
                        num_hbm.at[cid, pl.ds(sid * 632, 632)])

    @pl.when(sid == 15)
    def _dnum_last():
        pltpu.sync_copy(num_sh.at[pl.ds(9480, 520)],
                        num_hbm.at[cid, pl.ds(9480, 520)])

    @pl.when(sid < 10)
    def _dden():
        pltpu.sync_copy(den_sh.at[pl.ds(sid * 1000, 640)],
                        stage.at[pl.ds(0, 640)])
        pltpu.sync_copy(den_sh.at[pl.ds(sid * 1000 + 640, 360)],
                        zb1.at[pl.ds(0, 360)])
        pltpu.sync_copy(stage.at[pl.ds(0, 640)],
                        den_hbm.at[pl.ds(cid * N_NODES + sid * 1000, 640)])
        pltpu.sync_copy(zb1.at[pl.ds(0, 360)],
                        den_hbm.at[pl.ds(cid * N_NODES + sid * 1000 + 640,
                                         360)])


def _finish_body(num_ref, den_ref, b_ref, out_ref):
    s = num_ref[0] + num_ref[1]
    d = den_ref[0] + den_ref[1] + 1e-16
    out_ref[...] = jnp.maximum(s / d + b_ref[...], 0.0)


def _finish_call(num, den, b):
    return pl.pallas_call(
        _finish_body,
        out_shape=jax.ShapeDtypeStruct((N_NODES, OUT_CH), jnp.float32),
    )(num, den, b)


@jax.jit
def kernel(x, edge_index, W, att_src, att_dst, b):
    src = edge_index[0].astype(jnp.int32)
    dst = edge_index[1].astype(jnp.int32)
    h, a_s, a_d, gm16 = _prep_call(x, W, att_src.reshape(1, OUT_CH),
                                   att_dst.reshape(1, OUT_CH))
    num, den = _sc_edges(src, dst, a_s, a_d, gm16, h)
    out = _finish_call(num, den.reshape(NC, N_NODES, 1), b.reshape(1, OUT_CH))
    return out
